# Initial kernel scaffold; baseline (speedup 1.0000x reference)
import jax
import jax.numpy as jnp
from jax.experimental import pallas as pl


def kernel(x, adjacency_list, degree_list, W1s, W1n, b1, g1, be1, W2s, W2n, b2, g2, be2, Wd, bd):
    raise NotImplementedError("write your pallas kernel here")


# speedup vs baseline: 10.2238x; 10.2238x over previous
import functools

import jax
import jax.numpy as jnp
from jax import lax
from jax.experimental import pallas as pl
from jax.experimental.pallas import tpu as pltpu
from jax.experimental.pallas import tpu_sc as plsc

_N = 10000
_E = 320000
_D = 128
_T = 12
_NC = 2
_NS = 16
_NW = _NC * _NS
_EPW = _E // _NW
_CHB = 100
_NCH = _EPW // _CHB
_NPAD = 10240
_RPT = _NPAD // _NS
_ZR = 80
_BLK = 2000
_NBLK = _N // _BLK
_EPS = 1e-5


def _sc_agg_body(h_hbm, src_hbm, dst_hbm, out_hbm,
                 acc, sidx0, didx0, sidx1, didx1, rows0, rows1,
                 isem0, isem1, rsem0, rsem1):
    c = lax.axis_index("c")
    s = lax.axis_index("s")
    wid = c * _NS + s

    def idx_start(j, sbuf, dbuf, sem):
        pltpu.async_copy(src_hbm.at[wid, j], sbuf, sem)
        pltpu.async_copy(dst_hbm.at[wid, j], dbuf, sem)

    def idx_wait(j, sbuf, dbuf, sem):
        pltpu.make_async_copy(src_hbm.at[wid, j], sbuf, sem).wait()
        pltpu.make_async_copy(dst_hbm.at[wid, j], dbuf, sem).wait()

    z16 = jnp.zeros((16,), jnp.float32)

    def zstep(i, carry):
        rows0[i // 8, pl.ds((i % 8) * 16, 16)] = z16
        return carry

    lax.fori_loop(0, _ZR * 8, zstep, 0)

    def zcopy(k, carry):
        pltpu.sync_copy(rows0.at[pl.ds(0, _ZR)],
                        acc.at[pl.ds(s * _RPT + k * _ZR, _ZR)])
        return carry

    lax.fori_loop(0, _RPT // _ZR, zcopy, 0)
    plsc.subcore_barrier()

    idx_start(0, sidx0, didx0, isem0)
    idx_wait(0, sidx0, didx0, isem0)
    pltpu.async_copy(h_hbm.at[sidx0.at[0]], rows0, rsem0)
    idx_start(1, sidx1, didx1, isem1)

    def step(jj, carry):
        j0 = jj * 2
        j1 = j0 + 1
        j2 = j0 + 2
        j3 = j0 + 3
        idx_wait(j1, sidx1, didx1, isem1)
        pltpu.make_async_copy(h_hbm.at[sidx0.at[0]], rows0, rsem0).wait()
        pltpu.async_copy(h_hbm.at[sidx1.at[0]], rows1, rsem1)
        pltpu.sync_copy(rows0, acc.at[didx0.at[0]], add=True)

        @pl.when(j2 < _NCH)
        def _():
            idx_start(j2, sidx0, didx0, isem0)
            idx_wait(j2, sidx0, didx0, isem0)
            pltpu.async_copy(h_hbm.at[sidx0.at[0]], rows0, rsem0)

        pltpu.make_async_copy(h_hbm.at[sidx1.at[0]], rows1, rsem1).wait()

        @pl.when(j3 < _NCH)
        def _():
            idx_start(j3, sidx1, didx1, isem1)

        pltpu.sync_copy(rows1, acc.at[didx1.at[0]], add=True)
        return carry

    lax.fori_loop(0, _NCH // 2, step, 0)
    plsc.subcore_barrier()

    def rstep(k, carry):
        off = s * _RPT + k * _ZR
        pltpu.sync_copy(acc.at[pl.ds(off, _ZR)], rows0.at[pl.ds(0, _ZR)])
        pltpu.sync_copy(rows0.at[pl.ds(0, _ZR)], out_hbm.at[c, pl.ds(off, _ZR)])
        return carry

    lax.fori_loop(0, _RPT // _ZR, rstep, 0)


@functools.cache
def _build_sc_agg():
    return pl.kernel(
        _sc_agg_body,
        out_type=jax.ShapeDtypeStruct((_NC, _NPAD, _D), jnp.float32),
        mesh=plsc.VectorSubcoreMesh(core_axis_name="c", subcore_axis_name="s",
                                    num_cores=_NC, num_subcores=_NS),
        scratch_types=[
            pltpu.VMEM_SHARED((_NPAD, _D), jnp.float32),
            pltpu.VMEM((1, _CHB), jnp.int32),
            pltpu.VMEM((1, _CHB), jnp.int32),
            pltpu.VMEM((1, _CHB), jnp.int32),
            pltpu.VMEM((1, _CHB), jnp.int32),
            pltpu.VMEM((_CHB, _D), jnp.float32),
            pltpu.VMEM((_CHB, _D), jnp.float32),
            pltpu.SemaphoreType.DMA,
            pltpu.SemaphoreType.DMA,
            pltpu.SemaphoreType.DMA,
            pltpu.SemaphoreType.DMA,
        ],
    )


def _tc_pre_body(deg_ref, h_ref, agg_ref, ws_ref, wn_ref, b_ref,
                 pre_ref, stats_ref):
    i = pl.program_id(0)
    dinv = 1.0 / jnp.maximum(deg_ref[...], 1).astype(jnp.float32)
    a = (agg_ref[0] + agg_ref[1]) * dinv
    pre = (jnp.dot(h_ref[...], ws_ref[...], preferred_element_type=jnp.float32)
           + jnp.dot(a, wn_ref[...], preferred_element_type=jnp.float32)
           + b_ref[...])
    pre_ref[...] = pre
    st = jnp.concatenate(
        [jnp.sum(pre, axis=0)[None], jnp.sum(pre * pre, axis=0)[None]], axis=0)

    @pl.when(i == 0)
    def _():
        stats_ref[...] = st

    @pl.when(i > 0)
    def _():
        stats_ref[...] += st


_tc_pre = pl.pallas_call(
    _tc_pre_body,
    grid=(_NBLK,),
    in_specs=[
        pl.BlockSpec((_BLK, 1), lambda i: (i, 0)),
        pl.BlockSpec((_BLK, _D), lambda i: (i, 0)),
        pl.BlockSpec((_NC, _BLK, _D), lambda i: (0, i, 0)),
        pl.BlockSpec((_D, _D), lambda i: (0, 0)),
        pl.BlockSpec((_D, _D), lambda i: (0, 0)),
        pl.BlockSpec((1, _D), lambda i: (0, 0)),
    ],
    out_specs=[
        pl.BlockSpec((_BLK, _D), lambda i: (i, 0)),
        pl.BlockSpec((2, _D), lambda i: (0, 0)),
    ],
    out_shape=[
        jax.ShapeDtypeStruct((_N, _D), jnp.float32),
        jax.ShapeDtypeStruct((2, _D), jnp.float32),
    ],
)


def _tc_norm_body(stats_ref, g_ref, be_ref, pre_ref, out_ref):
    m = stats_ref[0:1, :] * (1.0 / _N)
    v = stats_ref[1:2, :] * (1.0 / _N) - m * m
    scale = lax.rsqrt(v + _EPS) * g_ref[...]
    out_ref[...] = jnp.maximum((pre_ref[...] - m) * scale + be_ref[...], 0.0)


_tc_norm = pl.pallas_call(
    _tc_norm_body,
    grid=(_NBLK,),
    in_specs=[
        pl.BlockSpec((2, _D), lambda i: (0, 0)),
        pl.BlockSpec((1, _D), lambda i: (0, 0)),
        pl.BlockSpec((1, _D), lambda i: (0, 0)),
        pl.BlockSpec((_BLK, _D), lambda i: (i, 0)),
    ],
    out_specs=pl.BlockSpec((_BLK, _D), lambda i: (i, 0)),
    out_shape=jax.ShapeDtypeStruct((_N, _D), jnp.float32),
)


def _tc_final_body(stats_ref, g_ref, be_ref, wd_ref, bd_ref, pre_ref,
                   out_ref, acc_ref):
    i = pl.program_id(0)
    m = stats_ref[0:1, :] * (1.0 / _N)
    v = stats_ref[1:2, :] * (1.0 / _N) - m * m
    scale = lax.rsqrt(v + _EPS) * g_ref[...]
    h = jnp.maximum((pre_ref[...] - m) * scale + be_ref[...], 0.0)
    cs = jnp.sum(h, axis=0)[None]

    @pl.when(i == 0)
    def _():
        acc_ref[...] = cs

    @pl.when(i > 0)
    def _():
        acc_ref[...] += cs

    @pl.when(i == _NBLK - 1)
    def _():
        out_ref[...] = (jnp.dot(acc_ref[...] * (1.0 / _N), wd_ref[...],
                                preferred_element_type=jnp.float32)
                        + bd_ref[...])


_tc_final = pl.pallas_call(
    _tc_final_body,
    grid=(_NBLK,),
    in_specs=[
        pl.BlockSpec((2, _D), lambda i: (0, 0)),
        pl.BlockSpec((1, _D), lambda i: (0, 0)),
        pl.BlockSpec((1, _D), lambda i: (0, 0)),
        pl.BlockSpec((_D, _T), lambda i: (0, 0)),
        pl.BlockSpec((1, _T), lambda i: (0, 0)),
        pl.BlockSpec((_BLK, _D), lambda i: (i, 0)),
    ],
    out_specs=pl.BlockSpec((1, _T), lambda i: (0, 0)),
    out_shape=jax.ShapeDtypeStruct((1, _T), jnp.float32),
    scratch_shapes=[pltpu.VMEM((1, _D), jnp.float32)],
)


def kernel(x, adjacency_list, degree_list,
           W1s, W1n, b1, g1, be1, W2s, W2n, b2, g2, be2, Wd, bd):
    src = adjacency_list[0].reshape(_NW, _NCH, 1, _CHB)
    dst = adjacency_list[1].reshape(_NW, _NCH, 1, _CHB)
    deg = degree_list.reshape(_N, 1)

    sc_agg = _build_sc_agg()
    agg1 = sc_agg(x, src, dst)
    pre1, st1 = _tc_pre(deg, x, agg1, W1s, W1n, b1.reshape(1, _D))
    h1 = _tc_norm(st1, g1.reshape(1, _D), be1.reshape(1, _D), pre1)

    agg2 = sc_agg(h1, src, dst)
    pre2, st2 = _tc_pre(deg, h1, agg2, W2s, W2n, b2.reshape(1, _D))
    out = _tc_final(st2, g2.reshape(1, _D), be2.reshape(1, _D),
                    Wd, bd.reshape(1, _T), pre2)
    return out.reshape(_T)
